# baseline (speedup 1.0000x reference)
import jax
import jax.numpy as jnp
from jax.experimental import pallas as pl
from jax.experimental.pallas import tpu as pltpu

_N = 16384
_C = 1000
_BLOCK_ROWS = 1024
_SCALE = 1.0 / (224.0 * 224.0)


def _body(x_ref, t_ref, out_ref, acc_sum, acc_cnt):
    i = pl.program_id(0)

    @pl.when(i == 0)
    def _init():
        acc_sum[...] = jnp.zeros_like(acc_sum)
        acc_cnt[...] = jnp.zeros_like(acc_cnt)

    x = x_ref[...]
    t = t_ref[...]
    d = x - t
    m = t > 0.0
    acc_sum[...] += jnp.sum(jnp.where(m, d * d, 0.0), axis=0, keepdims=True)
    acc_cnt[...] += jnp.sum(m.astype(jnp.float32), axis=0, keepdims=True)

    @pl.when(i == pl.num_programs(0) - 1)
    def _fin():
        s = acc_sum[...]
        c = acc_cnt[...]
        mean = jnp.where(c > 0.0, s / jnp.maximum(c, 1.0), 0.0)
        out_ref[0, 0] = jnp.sum(mean) * _SCALE


def kernel(inputs, targets):
    grid = (_N // _BLOCK_ROWS,)
    out = pl.pallas_call(
        _body,
        grid=grid,
        in_specs=[
            pl.BlockSpec((_BLOCK_ROWS, _C), lambda i: (i, 0)),
            pl.BlockSpec((_BLOCK_ROWS, _C), lambda i: (i, 0)),
        ],
        out_specs=pl.BlockSpec(memory_space=pltpu.SMEM),
        out_shape=jax.ShapeDtypeStruct((1, 1), jnp.float32),
        scratch_shapes=[
            pltpu.VMEM((1, _C), jnp.float32),
            pltpu.VMEM((1, _C), jnp.float32),
        ],
    )(inputs, targets)
    return out[0, 0]
